# baseline (device time: 56267 ns/iter reference)
import functools

import jax
import jax.numpy as jnp
from jax import lax
from jax.experimental import pallas as pl
from jax.experimental.pallas import tpu as pltpu

N_Z = 4
SCALE = 64 ** -0.5
BB = 2


def _body(q_ref, k_ref, v_ref, out_ref, comm_ref, send_sems, recv_sems,
          *, b, h, d):
    kk = k_ref.shape[-1]
    hd = h * d
    row_w = hd + 2 * h
    i = pl.program_id(0)

    my_x = lax.axis_index("x")
    my_y = lax.axis_index("y")
    my_z = lax.axis_index("z")

    @pl.when(i == 0)
    def _():
        barrier_sem = pltpu.get_barrier_semaphore()
        for dz in range(1, N_Z):
            pl.semaphore_signal(
                barrier_sem, inc=1,
                device_id=(my_x, my_y, lax.rem(my_z + dz, N_Z)),
                device_id_type=pl.DeviceIdType.MESH,
            )
        pl.semaphore_wait(barrier_sem, N_Z - 1)

    for bb in range(BB):
        ib = i * BB + bb
        q = q_ref[bb, 0]
        k2 = k_ref[bb].reshape(hd, kk)
        v2 = v_ref[bb].reshape(hd, kk)

        qflat = (q * SCALE).reshape(1, hd)
        qsel = (lax.broadcasted_iota(jnp.int32, (h, hd), 0)
                == lax.broadcasted_iota(jnp.int32, (h, hd), 1) // d)
        q2t = jnp.where(qsel, qflat, 0.0)

        st = jnp.dot(q2t, k2, preferred_element_type=jnp.float32)
        m = jnp.max(st, axis=1)
        p = jnp.exp(st - m[:, None])
        l = jnp.sum(p, axis=1)

        pt = p.T
        o_cross = jnp.dot(v2, pt,
                          preferred_element_type=jnp.float32)
        osel = (lax.broadcasted_iota(jnp.int32, (hd, h), 0) // d
                == lax.broadcasted_iota(jnp.int32, (hd, h), 1))
        o_flat = jnp.sum(jnp.where(osel, o_cross, 0.0), axis=1)

        row = jnp.concatenate(
            [o_flat[None, :], m[None, :], l[None, :]], axis=1)
        comm_ref[pl.ds(my_z, 1), pl.ds(ib, 1)] = row.reshape(1, 1, 1, row_w)

        for dz in range(1, N_Z):
            zz = lax.rem(my_z + dz, N_Z)
            rdma = pltpu.make_async_remote_copy(
                src_ref=comm_ref.at[my_z, ib],
                dst_ref=comm_ref.at[my_z, ib],
                send_sem=send_sems.at[dz - 1, ib],
                recv_sem=recv_sems.at[my_z, ib],
                device_id=(my_x, my_y, zz),
                device_id_type=pl.DeviceIdType.MESH,
            )
            rdma.start()

    @pl.when(i == b // BB - 1)
    def _():
        for dz in range(1, N_Z):
            src_z = lax.rem(my_z + dz, N_Z)
            for j in range(b):
                rdma = pltpu.make_async_remote_copy(
                    src_ref=comm_ref.at[src_z, j],
                    dst_ref=comm_ref.at[src_z, j],
                    send_sem=send_sems.at[dz - 1, j],
                    recv_sem=recv_sems.at[src_z, j],
                    device_id=(my_x, my_y, src_z),
                    device_id_type=pl.DeviceIdType.MESH,
                )
                rdma.wait_recv()

        c = comm_ref[...]
        cm = c[:, :, 0, hd:hd + h]
        cl = c[:, :, 0, hd + h:]
        co = c[..., :hd]
        g_m = jnp.max(cm, axis=0)
        alpha = jnp.exp(cm - g_m[None])
        g_l = jnp.sum(cl * alpha, axis=0)
        alpha_f = jnp.broadcast_to(
            alpha[..., None], (N_Z, b, h, d)).reshape(N_Z, b, 1, hd)
        g_l_f = jnp.broadcast_to(
            g_l[..., None], (b, h, d)).reshape(b, 1, hd)
        o = jnp.sum(co * alpha_f, axis=0) / g_l_f
        out_ref[...] = o.reshape(b, 1, h, d)

        for dz in range(1, N_Z):
            for j in range(b):
                rdma = pltpu.make_async_remote_copy(
                    src_ref=comm_ref.at[my_z, j],
                    dst_ref=comm_ref.at[my_z, j],
                    send_sem=send_sems.at[dz - 1, j],
                    recv_sem=recv_sems.at[my_z, j],
                    device_id=(my_x, my_y, lax.rem(my_z + dz, N_Z)),
                    device_id_type=pl.DeviceIdType.MESH,
                )
                rdma.wait_send()


def kernel(Q, K, V):
    b, kk, h, d = K.shape
    KT = jnp.transpose(K, (0, 2, 3, 1))
    VT = jnp.transpose(V, (0, 2, 3, 1))
    Q3 = Q.reshape(b, 1, h * d)
    row_w = h * d + 2 * h
    return pl.pallas_call(
        functools.partial(_body, b=b, h=h, d=d),
        grid=(b // BB,),
        in_specs=[
            pl.BlockSpec((BB, 1, h * d), lambda i: (i, 0, 0)),
            pl.BlockSpec((BB, h, d, kk), lambda i: (i, 0, 0, 0)),
            pl.BlockSpec((BB, h, d, kk), lambda i: (i, 0, 0, 0)),
        ],
        out_specs=pl.BlockSpec((b, 1, h, d), lambda i: (0, 0, 0, 0)),
        out_shape=jax.ShapeDtypeStruct((b, 1, h, d), jnp.float32),
        scratch_shapes=[
            pltpu.VMEM((N_Z, b, 1, row_w), jnp.float32),
            pltpu.SemaphoreType.DMA((N_Z - 1, b)),
            pltpu.SemaphoreType.DMA((N_Z, b)),
        ],
        compiler_params=pltpu.CompilerParams(
            has_side_effects=True,
            collective_id=0,
            vmem_limit_bytes=100 * 1024 * 1024,
        ),
    )(Q3, KT, VT)


# device time: 54123 ns/iter; 1.0396x vs baseline; 1.0396x over previous
import functools

import jax
import jax.numpy as jnp
from jax import lax
from jax.experimental import pallas as pl
from jax.experimental.pallas import tpu as pltpu

N_Z = 4
SCALE = 64 ** -0.5


def _body(q_ref, k_ref, v_ref, out_ref, comm_ref, send_sems, recv_sems,
          *, b, h, d):
    kk = k_ref.shape[-1]
    hd = h * d
    row_w = hd + 2 * h
    i = pl.program_id(0)

    my_x = lax.axis_index("x")
    my_y = lax.axis_index("y")
    my_z = lax.axis_index("z")

    @pl.when(i == 0)
    def _():
        barrier_sem = pltpu.get_barrier_semaphore()
        for dz in range(1, N_Z):
            pl.semaphore_signal(
                barrier_sem, inc=1,
                device_id=(my_x, my_y, lax.rem(my_z + dz, N_Z)),
                device_id_type=pl.DeviceIdType.MESH,
            )
        pl.semaphore_wait(barrier_sem, N_Z - 1)

    q = q_ref[0, 0]
    k2 = k_ref[0].reshape(hd, kk)
    v2 = v_ref[0].reshape(hd, kk)

    qflat = (q * SCALE).reshape(1, hd)
    qsel = (lax.broadcasted_iota(jnp.int32, (h, hd), 0)
            == lax.broadcasted_iota(jnp.int32, (h, hd), 1) // d)
    q2t = jnp.where(qsel, qflat, 0.0)

    st = jnp.dot(q2t, k2, preferred_element_type=jnp.float32)
    m = jnp.max(st, axis=1)
    p = jnp.exp(st - m[:, None])
    l = jnp.sum(p, axis=1)

    pt = p.T
    o_cross = jnp.dot(v2, pt, preferred_element_type=jnp.float32)
    osel = (lax.broadcasted_iota(jnp.int32, (hd, h), 0) // d
            == lax.broadcasted_iota(jnp.int32, (hd, h), 1))
    o_flat = jnp.sum(jnp.where(osel, o_cross, 0.0), axis=1)

    row = jnp.concatenate([o_flat[None, :], m[None, :], l[None, :]], axis=1)
    comm_ref[pl.ds(my_z, 1), pl.ds(i, 1)] = row.reshape(1, 1, 1, row_w)

    for dz in range(1, N_Z):
        zz = lax.rem(my_z + dz, N_Z)
        rdma = pltpu.make_async_remote_copy(
            src_ref=comm_ref.at[my_z, i],
            dst_ref=comm_ref.at[my_z, i],
            send_sem=send_sems.at[dz - 1, i],
            recv_sem=recv_sems.at[my_z, i],
            device_id=(my_x, my_y, zz),
            device_id_type=pl.DeviceIdType.MESH,
        )
        rdma.start()

    @pl.when(i == b - 1)
    def _():
        for dz in range(1, N_Z):
            src_z = lax.rem(my_z + dz, N_Z)
            for j in range(b):
                rdma = pltpu.make_async_remote_copy(
                    src_ref=comm_ref.at[src_z, j],
                    dst_ref=comm_ref.at[src_z, j],
                    send_sem=send_sems.at[dz - 1, j],
                    recv_sem=recv_sems.at[src_z, j],
                    device_id=(my_x, my_y, src_z),
                    device_id_type=pl.DeviceIdType.MESH,
                )
                rdma.wait_recv()

        c = comm_ref[...]
        cm = c[:, :, 0, hd:hd + h]
        cl = c[:, :, 0, hd + h:]
        co = c[..., :hd]
        g_m = jnp.max(cm, axis=0)
        alpha = jnp.exp(cm - g_m[None])
        g_l = jnp.sum(cl * alpha, axis=0)
        alpha_f = jnp.broadcast_to(
            alpha[..., None], (N_Z, b, h, d)).reshape(N_Z, b, 1, hd)
        g_l_f = jnp.broadcast_to(
            g_l[..., None], (b, h, d)).reshape(b, 1, hd)
        o = jnp.sum(co * alpha_f, axis=0) / g_l_f
        out_ref[...] = o.reshape(b, 1, h, d)

        for dz in range(1, N_Z):
            for j in range(b):
                rdma = pltpu.make_async_remote_copy(
                    src_ref=comm_ref.at[my_z, j],
                    dst_ref=comm_ref.at[my_z, j],
                    send_sem=send_sems.at[dz - 1, j],
                    recv_sem=recv_sems.at[my_z, j],
                    device_id=(my_x, my_y, lax.rem(my_z + dz, N_Z)),
                    device_id_type=pl.DeviceIdType.MESH,
                )
                rdma.wait_send()


def kernel(Q, K, V):
    b, kk, h, d = K.shape
    KT = jnp.transpose(K, (0, 2, 3, 1))
    VT = jnp.transpose(V, (0, 2, 3, 1))
    Q3 = Q.reshape(b, 1, h * d)
    row_w = h * d + 2 * h
    return pl.pallas_call(
        functools.partial(_body, b=b, h=h, d=d),
        grid=(b,),
        in_specs=[
            pl.BlockSpec((1, 1, h * d), lambda i: (i, 0, 0)),
            pl.BlockSpec((1, h, d, kk), lambda i: (i, 0, 0, 0)),
            pl.BlockSpec((1, h, d, kk), lambda i: (i, 0, 0, 0)),
        ],
        out_specs=pl.BlockSpec((b, 1, h, d), lambda i: (0, 0, 0, 0)),
        out_shape=jax.ShapeDtypeStruct((b, 1, h, d), jnp.float32),
        scratch_shapes=[
            pltpu.VMEM((N_Z, b, 1, row_w), jnp.float32),
            pltpu.SemaphoreType.DMA((N_Z - 1, b)),
            pltpu.SemaphoreType.DMA((N_Z, b)),
        ],
        compiler_params=pltpu.CompilerParams(
            has_side_effects=True,
            collective_id=0,
            vmem_limit_bytes=100 * 1024 * 1024,
        ),
    )(Q3, KT, VT)
